# baseline (speedup 1.0000x reference)
import jax
import jax.numpy as jnp
from jax import lax
from jax.experimental import pallas as pl
from jax.experimental.pallas import tpu as pltpu
from jax.experimental.pallas import tpu_sc as plsc

N = 10000
E = 320000
D_IN = 128
D_H = 32
D_OUT = 128
D = 128

NC = 2
NS = 16
K = 125
NCHUNK = E // (NC * NS * K)
N_PAD = 10240
DPT = N_PAD // NS

EPT = E // (NC * NS)
EPT_PAD = 10368
KS = 24
NCHS = EPT_PAD // KS
NB = 8

NP = 2
L = 5120
H_PAD = L + 128
HPT = H_PAD // NS
OPT = L // NS

_MESH = dict(core_axis_name="c", subcore_axis_name="s")


def _sc_deg_body(dst_hbm, ones_hbm, zer_hbm, out_hbm, didx, ones_v, dacc,
                 s0, s1, s2, s3):
    c = lax.axis_index("c")
    s = lax.axis_index("s")
    pltpu.sync_copy(dst_hbm.at[c, s], didx)
    pltpu.sync_copy(ones_hbm, ones_v)
    pltpu.sync_copy(zer_hbm, dacc.at[pl.ds(s * DPT, DPT)])
    plsc.subcore_barrier()

    sems = [s0, s1, s2, s3]
    for p in range(4):
        pltpu.async_copy(ones_v, dacc.at[didx.at[p]], sems[p], add=True)

    def step(t, carry):
        for p in range(4):
            jj = 4 * t + p
            pltpu.make_async_copy(ones_v, dacc.at[didx.at[jj - 4]],
                                  sems[p]).wait()
            pltpu.async_copy(ones_v, dacc.at[didx.at[jj]], sems[p], add=True)
        return carry

    lax.fori_loop(1, NCHUNK // 4, step, 0)
    for p in range(4):
        jj = NCHUNK - 4 + p
        pltpu.make_async_copy(ones_v, dacc.at[didx.at[jj]], sems[p]).wait()

    plsc.subcore_barrier()
    pltpu.sync_copy(dacc.at[pl.ds(s * DPT, DPT)],
                    out_hbm.at[c, pl.ds(s * DPT, DPT)])


def _sc_deg(dst4, ones_k, z_deg):
    return pl.kernel(
        _sc_deg_body,
        out_type=jax.ShapeDtypeStruct((NC, N_PAD), jnp.float32),
        mesh=plsc.VectorSubcoreMesh(**_MESH),
        scratch_types=[
            pltpu.VMEM((NCHUNK, K), jnp.int32),
            pltpu.VMEM((K,), jnp.float32),
            pltpu.VMEM_SHARED((N_PAD,), jnp.float32),
            pltpu.SemaphoreType.DMA,
            pltpu.SemaphoreType.DMA,
            pltpu.SemaphoreType.DMA,
            pltpu.SemaphoreType.DMA,
        ],
        name="sc_gcn_deg",
    )(dst4, ones_k, z_deg)


def _sc_scatter_body(y_hbm, src_hbm, dst_hbm, zer_hbm, out_hbm,
                     sidx, didx,
                     b0, b1, b2, b3, b4, b5, b6, b7, acc,
                     g0, g1, g2, g3, g4, g5, g6, g7,
                     s0, s1, s2, s3, s4, s5, s6, s7):
    c = lax.axis_index("c")
    s = lax.axis_index("s")
    pltpu.sync_copy(src_hbm.at[c, s], sidx)

    bufs = [b0, b1, b2, b3, b4, b5, b6, b7]
    gs = [g0, g1, g2, g3, g4, g5, g6, g7]
    ss = [s0, s1, s2, s3, s4, s5, s6, s7]

    def g_start(jj, u):
        pltpu.async_copy(y_hbm.at[sidx.at[pl.ds(jj * KS, KS)]], bufs[u], gs[u])

    def g_wait(jj, u):
        pltpu.make_async_copy(y_hbm.at[sidx.at[pl.ds(jj * KS, KS)]], bufs[u], gs[u]).wait()

    def s_start(jj, u):
        pltpu.async_copy(bufs[u], acc.at[didx.at[pl.ds(jj * KS, KS)]], ss[u], add=True)

    def s_wait(jj, u):
        pltpu.make_async_copy(bufs[u], acc.at[didx.at[pl.ds(jj * KS, KS)]], ss[u]).wait()

    pltpu.sync_copy(dst_hbm.at[c, s], didx)
    pltpu.sync_copy(zer_hbm, acc.at[pl.ds(s * HPT, HPT), :])
    plsc.subcore_barrier()

    for u in range(4):
        g_start(u, u)
    for u in range(8):
        g_wait(u, u)
        s_start(u, u)
        if u >= 4:
            s_wait(u - 4, u - 4)
        g_start(u + 4, (u + 4) % NB)

    def step(t, carry):
        for u in range(8):
            jj = NB * t + u
            g_wait(jj, u)
            s_start(jj, u)
            s_wait(jj - 4, (u + 4) % NB)

            @pl.when(jj + 4 < NCHS)
            def _():
                g_start(jj + 4, (u + 4) % NB)
        return carry

    lax.fori_loop(1, NCHS // NB, step, 0)
    for u in range(4, 8):
        s_wait(NCHS - NB + u, u)
    plsc.subcore_barrier()
    pltpu.sync_copy(acc.at[pl.ds(s * OPT, OPT), :],
                    out_hbm.at[c, pl.ds(s * OPT, OPT), :])


def _sc_scatter(y, src4, dst4h, zrows):
    return pl.kernel(
        _sc_scatter_body,
        out_type=jax.ShapeDtypeStruct((NC, L, D), jnp.float32),
        mesh=plsc.VectorSubcoreMesh(**_MESH),
        scratch_types=(
            [pltpu.VMEM((EPT_PAD,), jnp.int32),
             pltpu.VMEM((EPT_PAD,), jnp.int32)]
            + [pltpu.VMEM((KS, D), jnp.float32) for _ in range(NB)]
            + [pltpu.VMEM_SHARED((H_PAD, D), jnp.float32)]
            + [pltpu.SemaphoreType.DMA for _ in range(2 * NB)]
        ),
        name="sc_gcn_scatter",
    )(y, src4, dst4h, zrows)


R = 1000


def _tc_prep_body(x_ref, d0_ref, d1_ref, g_ref, dinv_ref):
    dinv = lax.rsqrt(d0_ref[...] + d1_ref[...] + 1.0)
    g_ref[...] = x_ref[...] * dinv
    dinv_ref[...] = dinv


def _tc_prep(x, d0, d1):
    return pl.pallas_call(
        _tc_prep_body,
        grid=(N // R,),
        in_specs=[
            pl.BlockSpec((R, D_IN), lambda i: (i, 0)),
            pl.BlockSpec((R, 1), lambda i: (i, 0)),
            pl.BlockSpec((R, 1), lambda i: (i, 0)),
        ],
        out_specs=[
            pl.BlockSpec((R, D_IN), lambda i: (i, 0)),
            pl.BlockSpec((R, 1), lambda i: (i, 0)),
        ],
        out_shape=[
            jax.ShapeDtypeStruct((N, D_IN), jnp.float32),
            jax.ShapeDtypeStruct((N, 1), jnp.float32),
        ],
    )(x, d0, d1)


def _tc_mid_body(acc_ref, g_ref, dinv_ref, b0_ref, w0_ref, w1_ref, y1_ref):
    dinv = dinv_ref[...]
    agg = acc_ref[0] + acc_ref[1] + g_ref[...]
    pre = jnp.dot(agg, w0_ref[...],
                  preferred_element_type=jnp.float32) * dinv + b0_ref[...]
    h = jnp.maximum(pre, 0.0)
    y1_ref[...] = jnp.dot(h, w1_ref[...],
                          preferred_element_type=jnp.float32) * dinv


def _tc_mid(accp, g, dinv, b0, w0, w1):
    return pl.pallas_call(
        _tc_mid_body,
        grid=(N // R,),
        in_specs=[
            pl.BlockSpec((NC, R, D_IN), lambda i: (0, i, 0)),
            pl.BlockSpec((R, D_IN), lambda i: (i, 0)),
            pl.BlockSpec((R, 1), lambda i: (i, 0)),
            pl.BlockSpec((1, D_H), lambda i: (0, 0)),
            pl.BlockSpec((D_IN, D_H), lambda i: (0, 0)),
            pl.BlockSpec((D_H, D_OUT), lambda i: (0, 0)),
        ],
        out_specs=pl.BlockSpec((R, D_OUT), lambda i: (i, 0)),
        out_shape=jax.ShapeDtypeStruct((N, D_OUT), jnp.float32),
    )(accp, g, dinv, b0, w0, w1)


def _tc_final_body(acc_ref, y1_ref, dinv_ref, b1_ref, out_ref):
    pre = (acc_ref[0] + acc_ref[1] + y1_ref[...]) * dinv_ref[...] + b1_ref[...]
    out_ref[...] = jnp.maximum(pre, 0.0)


def _tc_final(accp, y1, dinv, b1):
    return pl.pallas_call(
        _tc_final_body,
        grid=(N // R,),
        in_specs=[
            pl.BlockSpec((NC, R, D_OUT), lambda i: (0, i, 0)),
            pl.BlockSpec((R, D_OUT), lambda i: (i, 0)),
            pl.BlockSpec((R, 1), lambda i: (i, 0)),
            pl.BlockSpec((1, D_OUT), lambda i: (0, 0)),
        ],
        out_specs=pl.BlockSpec((R, D_OUT), lambda i: (i, 0)),
        out_shape=jax.ShapeDtypeStruct((N, D_OUT), jnp.float32),
    )(accp, y1, dinv, b1)


@jax.jit
def kernel(x, edge_index, W0, b0, W1, b1):
    src = edge_index[0].astype(jnp.int32)
    dst = edge_index[1].astype(jnp.int32)
    pad_width = ((0, 0), (0, 0), (0, EPT_PAD - EPT))
    src4 = jnp.pad(src.reshape(NC, NS, EPT), pad_width)
    dst4 = dst.reshape(NC, NS, NCHUNK, K)
    fake = N_PAD + (jnp.arange(EPT_PAD - EPT, dtype=jnp.int32) % 128)
    dst3 = jnp.pad(dst.reshape(NC, NS, EPT), pad_width)
    dst3 = dst3.at[:, :, EPT:].set(fake[None, None, :])
    dump = L + (dst3 & 127)
    dstP = [jnp.where((dst3 >= p * L) & (dst3 < (p + 1) * L), dst3 - p * L,
                      dump) for p in range(NP)]
    ones_k = jnp.ones((K,), jnp.float32)
    z_deg = jnp.zeros((DPT,), jnp.float32)
    zrows = jnp.zeros((HPT, D), jnp.float32)

    degp = _sc_deg(dst4, ones_k, z_deg)
    d0 = degp[0, :N].reshape(N, 1)
    d1 = degp[1, :N].reshape(N, 1)

    g, dinv = _tc_prep(x, d0, d1)
    accp0 = jnp.concatenate(
        [_sc_scatter(g, src4, dstP[p], zrows) for p in range(NP)], axis=1)
    y1 = _tc_mid(accp0, g, dinv, b0.reshape(1, D_H), W0, W1)
    accp1 = jnp.concatenate(
        [_sc_scatter(y1, src4, dstP[p], zrows) for p in range(NP)], axis=1)
    return _tc_final(accp1, y1, dinv, b1.reshape(1, D_OUT))


# speedup vs baseline: 1.1288x; 1.1288x over previous
import jax
import jax.numpy as jnp
from jax import lax
from jax.experimental import pallas as pl
from jax.experimental.pallas import tpu as pltpu
from jax.experimental.pallas import tpu_sc as plsc

N = 10000
E = 320000
D_IN = 128
D_H = 32
D_OUT = 128
D = 128

NC = 2
NS = 16
K = 125
NCHUNK = E // (NC * NS * K)
N_PAD = 10240
DPT = N_PAD // NS

EPT = E // (NC * NS)
EPT_PAD = 10240
KS = 8
NCHS = EPT_PAD // KS
NB = 8

NP = 2
L = 5120
H_PAD = L + 128
HPT = H_PAD // NS
OPT = L // NS

_MESH = dict(core_axis_name="c", subcore_axis_name="s")


def _sc_deg_body(dst_hbm, ones_hbm, zer_hbm, out_hbm, didx, ones_v, dacc,
                 s0, s1, s2, s3):
    c = lax.axis_index("c")
    s = lax.axis_index("s")
    pltpu.sync_copy(dst_hbm.at[c, s], didx)
    pltpu.sync_copy(ones_hbm, ones_v)
    pltpu.sync_copy(zer_hbm, dacc.at[pl.ds(s * DPT, DPT)])
    plsc.subcore_barrier()

    sems = [s0, s1, s2, s3]
    for p in range(4):
        pltpu.async_copy(ones_v, dacc.at[didx.at[p]], sems[p], add=True)

    def step(t, carry):
        for p in range(4):
            jj = 4 * t + p
            pltpu.make_async_copy(ones_v, dacc.at[didx.at[jj - 4]],
                                  sems[p]).wait()
            pltpu.async_copy(ones_v, dacc.at[didx.at[jj]], sems[p], add=True)
        return carry

    lax.fori_loop(1, NCHUNK // 4, step, 0)
    for p in range(4):
        jj = NCHUNK - 4 + p
        pltpu.make_async_copy(ones_v, dacc.at[didx.at[jj]], sems[p]).wait()

    plsc.subcore_barrier()
    pltpu.sync_copy(dacc.at[pl.ds(s * DPT, DPT)],
                    out_hbm.at[c, pl.ds(s * DPT, DPT)])


def _sc_deg(dst4, ones_k, z_deg):
    return pl.kernel(
        _sc_deg_body,
        out_type=jax.ShapeDtypeStruct((NC, N_PAD), jnp.float32),
        mesh=plsc.VectorSubcoreMesh(**_MESH),
        scratch_types=[
            pltpu.VMEM((NCHUNK, K), jnp.int32),
            pltpu.VMEM((K,), jnp.float32),
            pltpu.VMEM_SHARED((N_PAD,), jnp.float32),
            pltpu.SemaphoreType.DMA,
            pltpu.SemaphoreType.DMA,
            pltpu.SemaphoreType.DMA,
            pltpu.SemaphoreType.DMA,
        ],
        name="sc_gcn_deg",
    )(dst4, ones_k, z_deg)


def _sc_scatter_body(y_hbm, src_hbm, dst_hbm, zer_hbm, out_hbm,
                     sidx, didx,
                     b0, b1, b2, b3, b4, b5, b6, b7, acc,
                     g0, g1, g2, g3, g4, g5, g6, g7,
                     s0, s1, s2, s3, s4, s5, s6, s7):
    c = lax.axis_index("c")
    s = lax.axis_index("s")
    pltpu.sync_copy(src_hbm.at[c, s], sidx)

    bufs = [b0, b1, b2, b3, b4, b5, b6, b7]
    gs = [g0, g1, g2, g3, g4, g5, g6, g7]
    ss = [s0, s1, s2, s3, s4, s5, s6, s7]

    def g_start(jj, u):
        pltpu.async_copy(y_hbm.at[sidx.at[pl.ds(jj * KS, KS)]], bufs[u], gs[u])

    def g_wait(jj, u):
        pltpu.make_async_copy(y_hbm.at[sidx.at[pl.ds(jj * KS, KS)]], bufs[u], gs[u]).wait()

    def s_start(jj, u):
        pltpu.async_copy(bufs[u], acc.at[didx.at[pl.ds(jj * KS, KS)]], ss[u], add=True)

    def s_wait(jj, u):
        pltpu.make_async_copy(bufs[u], acc.at[didx.at[pl.ds(jj * KS, KS)]], ss[u]).wait()

    pltpu.sync_copy(dst_hbm.at[c, s], didx)
    pltpu.sync_copy(zer_hbm, acc.at[pl.ds(s * HPT, HPT), :])
    plsc.subcore_barrier()

    for u in range(4):
        g_start(u, u)
    for u in range(8):
        g_wait(u, u)
        s_start(u, u)
        if u >= 4:
            s_wait(u - 4, u - 4)
        g_start(u + 4, (u + 4) % NB)

    def step(t, carry):
        for u in range(8):
            jj = NB * t + u
            g_wait(jj, u)
            s_start(jj, u)
            s_wait(jj - 4, (u + 4) % NB)

            @pl.when(jj + 4 < NCHS)
            def _():
                g_start(jj + 4, (u + 4) % NB)
        return carry

    lax.fori_loop(1, NCHS // NB, step, 0)
    for u in range(4, 8):
        s_wait(NCHS - NB + u, u)
    plsc.subcore_barrier()
    pltpu.sync_copy(acc.at[pl.ds(s * OPT, OPT), :],
                    out_hbm.at[c, pl.ds(s * OPT, OPT), :])


def _sc_scatter(y, src4, dst4h, zrows):
    return pl.kernel(
        _sc_scatter_body,
        out_type=jax.ShapeDtypeStruct((NC, L, D), jnp.float32),
        mesh=plsc.VectorSubcoreMesh(**_MESH),
        scratch_types=(
            [pltpu.VMEM((EPT_PAD,), jnp.int32),
             pltpu.VMEM((EPT_PAD,), jnp.int32)]
            + [pltpu.VMEM((KS, D), jnp.float32) for _ in range(NB)]
            + [pltpu.VMEM_SHARED((H_PAD, D), jnp.float32)]
            + [pltpu.SemaphoreType.DMA for _ in range(2 * NB)]
        ),
        name="sc_gcn_scatter",
    )(y, src4, dst4h, zrows)


R = 1000


def _tc_prep_body(x_ref, d0_ref, d1_ref, g_ref, dinv_ref):
    dinv = lax.rsqrt(d0_ref[...] + d1_ref[...] + 1.0)
    g_ref[...] = x_ref[...] * dinv
    dinv_ref[...] = dinv


def _tc_prep(x, d0, d1):
    return pl.pallas_call(
        _tc_prep_body,
        grid=(N // R,),
        in_specs=[
            pl.BlockSpec((R, D_IN), lambda i: (i, 0)),
            pl.BlockSpec((R, 1), lambda i: (i, 0)),
            pl.BlockSpec((R, 1), lambda i: (i, 0)),
        ],
        out_specs=[
            pl.BlockSpec((R, D_IN), lambda i: (i, 0)),
            pl.BlockSpec((R, 1), lambda i: (i, 0)),
        ],
        out_shape=[
            jax.ShapeDtypeStruct((N, D_IN), jnp.float32),
            jax.ShapeDtypeStruct((N, 1), jnp.float32),
        ],
    )(x, d0, d1)


def _tc_mid_body(acc_ref, g_ref, dinv_ref, b0_ref, w0_ref, w1_ref, y1_ref):
    dinv = dinv_ref[...]
    agg = acc_ref[0] + acc_ref[1] + g_ref[...]
    pre = jnp.dot(agg, w0_ref[...],
                  preferred_element_type=jnp.float32) * dinv + b0_ref[...]
    h = jnp.maximum(pre, 0.0)
    y1_ref[...] = jnp.dot(h, w1_ref[...],
                          preferred_element_type=jnp.float32) * dinv


def _tc_mid(accp, g, dinv, b0, w0, w1):
    return pl.pallas_call(
        _tc_mid_body,
        grid=(N // R,),
        in_specs=[
            pl.BlockSpec((NC, R, D_IN), lambda i: (0, i, 0)),
            pl.BlockSpec((R, D_IN), lambda i: (i, 0)),
            pl.BlockSpec((R, 1), lambda i: (i, 0)),
            pl.BlockSpec((1, D_H), lambda i: (0, 0)),
            pl.BlockSpec((D_IN, D_H), lambda i: (0, 0)),
            pl.BlockSpec((D_H, D_OUT), lambda i: (0, 0)),
        ],
        out_specs=pl.BlockSpec((R, D_OUT), lambda i: (i, 0)),
        out_shape=jax.ShapeDtypeStruct((N, D_OUT), jnp.float32),
    )(accp, g, dinv, b0, w0, w1)


def _tc_final_body(acc_ref, y1_ref, dinv_ref, b1_ref, out_ref):
    pre = (acc_ref[0] + acc_ref[1] + y1_ref[...]) * dinv_ref[...] + b1_ref[...]
    out_ref[...] = jnp.maximum(pre, 0.0)


def _tc_final(accp, y1, dinv, b1):
    return pl.pallas_call(
        _tc_final_body,
        grid=(N // R,),
        in_specs=[
            pl.BlockSpec((NC, R, D_OUT), lambda i: (0, i, 0)),
            pl.BlockSpec((R, D_OUT), lambda i: (i, 0)),
            pl.BlockSpec((R, 1), lambda i: (i, 0)),
            pl.BlockSpec((1, D_OUT), lambda i: (0, 0)),
        ],
        out_specs=pl.BlockSpec((R, D_OUT), lambda i: (i, 0)),
        out_shape=jax.ShapeDtypeStruct((N, D_OUT), jnp.float32),
    )(accp, y1, dinv, b1)


@jax.jit
def kernel(x, edge_index, W0, b0, W1, b1):
    src = edge_index[0].astype(jnp.int32)
    dst = edge_index[1].astype(jnp.int32)
    pad_width = ((0, 0), (0, 0), (0, EPT_PAD - EPT))
    src4 = jnp.pad(src.reshape(NC, NS, EPT), pad_width)
    dst4 = dst.reshape(NC, NS, NCHUNK, K)
    fake = N_PAD + (jnp.arange(EPT_PAD - EPT, dtype=jnp.int32) % 128)
    dst3 = jnp.pad(dst.reshape(NC, NS, EPT), pad_width)
    dst3 = dst3.at[:, :, EPT:].set(fake[None, None, :])
    dump = L + (dst3 & 127)
    dstP = [jnp.where((dst3 >= p * L) & (dst3 < (p + 1) * L), dst3 - p * L,
                      dump) for p in range(NP)]
    ones_k = jnp.ones((K,), jnp.float32)
    z_deg = jnp.zeros((DPT,), jnp.float32)
    zrows = jnp.zeros((HPT, D), jnp.float32)

    degp = _sc_deg(dst4, ones_k, z_deg)
    d0 = degp[0, :N].reshape(N, 1)
    d1 = degp[1, :N].reshape(N, 1)

    g, dinv = _tc_prep(x, d0, d1)
    accp0 = jnp.concatenate(
        [_sc_scatter(g, src4, dstP[p], zrows) for p in range(NP)], axis=1)
    y1 = _tc_mid(accp0, g, dinv, b0.reshape(1, D_H), W0, W1)
    accp1 = jnp.concatenate(
        [_sc_scatter(y1, src4, dstP[p], zrows) for p in range(NP)], axis=1)
    return _tc_final(accp1, y1, dinv, b1.reshape(1, D_OUT))


# speedup vs baseline: 1.3041x; 1.1553x over previous
import jax
import jax.numpy as jnp
from jax import lax
from jax.experimental import pallas as pl
from jax.experimental.pallas import tpu as pltpu
from jax.experimental.pallas import tpu_sc as plsc

N = 10000
E = 320000
D_IN = 128
D_H = 32
D_OUT = 128
D = 128

NC = 2
NS = 16
K = 125
NCHUNK = E // (NC * NS * K)
N_PAD = 10240
DPT = N_PAD // NS

EPT = E // (NC * NS)
EPT_PAD = 10240
KS = 16
NCHS = EPT_PAD // KS
NB = 10
GD = 4
SD = NB - GD

NP = 2
L = 5120
H_PAD = L + 128
HPT = H_PAD // NS
OPT = L // NS

_MESH = dict(core_axis_name="c", subcore_axis_name="s")


def _sc_deg_body(dst_hbm, ones_hbm, zer_hbm, out_hbm, didx, ones_v, dacc,
                 s0, s1, s2, s3):
    c = lax.axis_index("c")
    s = lax.axis_index("s")
    pltpu.sync_copy(dst_hbm.at[c, s], didx)
    pltpu.sync_copy(ones_hbm, ones_v)
    pltpu.sync_copy(zer_hbm, dacc.at[pl.ds(s * DPT, DPT)])
    plsc.subcore_barrier()

    sems = [s0, s1, s2, s3]
    for p in range(4):
        pltpu.async_copy(ones_v, dacc.at[didx.at[p]], sems[p], add=True)

    def step(t, carry):
        for p in range(4):
            jj = 4 * t + p
            pltpu.make_async_copy(ones_v, dacc.at[didx.at[jj - 4]],
                                  sems[p]).wait()
            pltpu.async_copy(ones_v, dacc.at[didx.at[jj]], sems[p], add=True)
        return carry

    lax.fori_loop(1, NCHUNK // 4, step, 0)
    for p in range(4):
        jj = NCHUNK - 4 + p
        pltpu.make_async_copy(ones_v, dacc.at[didx.at[jj]], sems[p]).wait()

    plsc.subcore_barrier()
    pltpu.sync_copy(dacc.at[pl.ds(s * DPT, DPT)],
                    out_hbm.at[c, pl.ds(s * DPT, DPT)])


def _sc_deg(dst4, ones_k, z_deg):
    return pl.kernel(
        _sc_deg_body,
        out_type=jax.ShapeDtypeStruct((NC, N_PAD), jnp.float32),
        mesh=plsc.VectorSubcoreMesh(**_MESH),
        scratch_types=[
            pltpu.VMEM((NCHUNK, K), jnp.int32),
            pltpu.VMEM((K,), jnp.float32),
            pltpu.VMEM_SHARED((N_PAD,), jnp.float32),
            pltpu.SemaphoreType.DMA,
            pltpu.SemaphoreType.DMA,
            pltpu.SemaphoreType.DMA,
            pltpu.SemaphoreType.DMA,
        ],
        name="sc_gcn_deg",
    )(dst4, ones_k, z_deg)


def _sc_scatter_body(y_hbm, src_hbm, dst_hbm, zer_hbm, out_hbm,
                     sidx, didx, *rest):
    c = lax.axis_index("c")
    s = lax.axis_index("s")
    pltpu.sync_copy(src_hbm.at[c, s], sidx)

    bufs = list(rest[:NB])
    acc = rest[NB]
    gs = list(rest[NB + 1:2 * NB + 1])
    ss = list(rest[2 * NB + 1:3 * NB + 1])

    def g_start(jj, u):
        pltpu.async_copy(y_hbm.at[sidx.at[pl.ds(jj * KS, KS)]], bufs[u], gs[u])

    def g_wait(jj, u):
        pltpu.make_async_copy(y_hbm.at[sidx.at[pl.ds(jj * KS, KS)]], bufs[u], gs[u]).wait()

    def s_start(jj, u):
        pltpu.async_copy(bufs[u], acc.at[didx.at[pl.ds(jj * KS, KS)]], ss[u], add=True)

    def s_wait(jj, u):
        pltpu.make_async_copy(bufs[u], acc.at[didx.at[pl.ds(jj * KS, KS)]], ss[u]).wait()

    pltpu.sync_copy(dst_hbm.at[c, s], didx)
    pltpu.sync_copy(zer_hbm, acc.at[pl.ds(s * HPT, HPT), :])
    plsc.subcore_barrier()

    for u in range(GD):
        g_start(u, u)
    for u in range(NB):
        g_wait(u, u)
        s_start(u, u)
        if u >= SD:
            s_wait(u - SD, u - SD)
        g_start(u + GD, (u + GD) % NB)

    def step(t, carry):
        for u in range(NB):
            jj = NB * t + u
            g_wait(jj, u)
            s_start(jj, u)
            s_wait(jj - SD, (u + GD) % NB)

            @pl.when(jj + GD < NCHS)
            def _():
                g_start(jj + GD, (u + GD) % NB)
        return carry

    lax.fori_loop(1, NCHS // NB, step, 0)
    for u in range(NB - SD, NB):
        s_wait(NCHS - NB + u, u)
    plsc.subcore_barrier()
    pltpu.sync_copy(acc.at[pl.ds(s * OPT, OPT), :],
                    out_hbm.at[c, pl.ds(s * OPT, OPT), :])


def _sc_scatter(y, src4, dst4h, zrows):
    return pl.kernel(
        _sc_scatter_body,
        out_type=jax.ShapeDtypeStruct((NC, L, D), jnp.float32),
        mesh=plsc.VectorSubcoreMesh(**_MESH),
        scratch_types=(
            [pltpu.VMEM((EPT_PAD,), jnp.int32),
             pltpu.VMEM((EPT_PAD,), jnp.int32)]
            + [pltpu.VMEM((KS, D), jnp.float32) for _ in range(NB)]
            + [pltpu.VMEM_SHARED((H_PAD, D), jnp.float32)]
            + [pltpu.SemaphoreType.DMA for _ in range(2 * NB)]
        ),
        name="sc_gcn_scatter",
    )(y, src4, dst4h, zrows)


R = 1000


def _tc_prep_body(x_ref, d0_ref, d1_ref, g_ref, dinv_ref):
    dinv = lax.rsqrt(d0_ref[...] + d1_ref[...] + 1.0)
    g_ref[...] = x_ref[...] * dinv
    dinv_ref[...] = dinv


def _tc_prep(x, d0, d1):
    return pl.pallas_call(
        _tc_prep_body,
        grid=(N // R,),
        in_specs=[
            pl.BlockSpec((R, D_IN), lambda i: (i, 0)),
            pl.BlockSpec((R, 1), lambda i: (i, 0)),
            pl.BlockSpec((R, 1), lambda i: (i, 0)),
        ],
        out_specs=[
            pl.BlockSpec((R, D_IN), lambda i: (i, 0)),
            pl.BlockSpec((R, 1), lambda i: (i, 0)),
        ],
        out_shape=[
            jax.ShapeDtypeStruct((N, D_IN), jnp.float32),
            jax.ShapeDtypeStruct((N, 1), jnp.float32),
        ],
    )(x, d0, d1)


def _tc_mid_body(acc_ref, g_ref, dinv_ref, b0_ref, w0_ref, w1_ref, y1_ref):
    dinv = dinv_ref[...]
    agg = acc_ref[0] + acc_ref[1] + g_ref[...]
    pre = jnp.dot(agg, w0_ref[...],
                  preferred_element_type=jnp.float32) * dinv + b0_ref[...]
    h = jnp.maximum(pre, 0.0)
    y1_ref[...] = jnp.dot(h, w1_ref[...],
                          preferred_element_type=jnp.float32) * dinv


def _tc_mid(accp, g, dinv, b0, w0, w1):
    return pl.pallas_call(
        _tc_mid_body,
        grid=(N // R,),
        in_specs=[
            pl.BlockSpec((NC, R, D_IN), lambda i: (0, i, 0)),
            pl.BlockSpec((R, D_IN), lambda i: (i, 0)),
            pl.BlockSpec((R, 1), lambda i: (i, 0)),
            pl.BlockSpec((1, D_H), lambda i: (0, 0)),
            pl.BlockSpec((D_IN, D_H), lambda i: (0, 0)),
            pl.BlockSpec((D_H, D_OUT), lambda i: (0, 0)),
        ],
        out_specs=pl.BlockSpec((R, D_OUT), lambda i: (i, 0)),
        out_shape=jax.ShapeDtypeStruct((N, D_OUT), jnp.float32),
    )(accp, g, dinv, b0, w0, w1)


def _tc_final_body(acc_ref, y1_ref, dinv_ref, b1_ref, out_ref):
    pre = (acc_ref[0] + acc_ref[1] + y1_ref[...]) * dinv_ref[...] + b1_ref[...]
    out_ref[...] = jnp.maximum(pre, 0.0)


def _tc_final(accp, y1, dinv, b1):
    return pl.pallas_call(
        _tc_final_body,
        grid=(N // R,),
        in_specs=[
            pl.BlockSpec((NC, R, D_OUT), lambda i: (0, i, 0)),
            pl.BlockSpec((R, D_OUT), lambda i: (i, 0)),
            pl.BlockSpec((R, 1), lambda i: (i, 0)),
            pl.BlockSpec((1, D_OUT), lambda i: (0, 0)),
        ],
        out_specs=pl.BlockSpec((R, D_OUT), lambda i: (i, 0)),
        out_shape=jax.ShapeDtypeStruct((N, D_OUT), jnp.float32),
    )(accp, y1, dinv, b1)


@jax.jit
def kernel(x, edge_index, W0, b0, W1, b1):
    src = edge_index[0].astype(jnp.int32)
    dst = edge_index[1].astype(jnp.int32)
    pad_width = ((0, 0), (0, 0), (0, EPT_PAD - EPT))
    src4 = jnp.pad(src.reshape(NC, NS, EPT), pad_width)
    dst4 = dst.reshape(NC, NS, NCHUNK, K)
    fake = N_PAD + (jnp.arange(EPT_PAD - EPT, dtype=jnp.int32) % 128)
    dst3 = jnp.pad(dst.reshape(NC, NS, EPT), pad_width)
    dst3 = dst3.at[:, :, EPT:].set(fake[None, None, :])
    dump = L + (dst3 & 127)
    dstP = [jnp.where((dst3 >= p * L) & (dst3 < (p + 1) * L), dst3 - p * L,
                      dump) for p in range(NP)]
    ones_k = jnp.ones((K,), jnp.float32)
    z_deg = jnp.zeros((DPT,), jnp.float32)
    zrows = jnp.zeros((HPT, D), jnp.float32)

    degp = _sc_deg(dst4, ones_k, z_deg)
    d0 = degp[0, :N].reshape(N, 1)
    d1 = degp[1, :N].reshape(N, 1)

    g, dinv = _tc_prep(x, d0, d1)
    accp0 = jnp.concatenate(
        [_sc_scatter(g, src4, dstP[p], zrows) for p in range(NP)], axis=1)
    y1 = _tc_mid(accp0, g, dinv, b0.reshape(1, D_H), W0, W1)
    accp1 = jnp.concatenate(
        [_sc_scatter(y1, src4, dstP[p], zrows) for p in range(NP)], axis=1)
    return _tc_final(accp1, y1, dinv, b1.reshape(1, D_OUT))


# speedup vs baseline: 1.3049x; 1.0006x over previous
import jax
import jax.numpy as jnp
from jax import lax
from jax.experimental import pallas as pl
from jax.experimental.pallas import tpu as pltpu
from jax.experimental.pallas import tpu_sc as plsc

N = 10000
E = 320000
D_IN = 128
D_H = 32
D_OUT = 128
D = 128

NC = 2
NS = 16
K = 125
NCHUNK = E // (NC * NS * K)
N_PAD = 10240
DPT = N_PAD // NS

EPT = E // (NC * NS)
EPT_PAD = 10240
KS = 16
NCHS = EPT_PAD // KS
NB = 8

NP = 2
L = 5120
H_PAD = L + 128
HPT = H_PAD // NS
OPT = L // NS

_MESH = dict(core_axis_name="c", subcore_axis_name="s")


def _sc_deg_body(dst_hbm, ones_hbm, zer_hbm, out_hbm, didx, ones_v, dacc,
                 s0, s1, s2, s3):
    c = lax.axis_index("c")
    s = lax.axis_index("s")
    pltpu.sync_copy(dst_hbm.at[c, s], didx)
    pltpu.sync_copy(ones_hbm, ones_v)
    pltpu.sync_copy(zer_hbm, dacc.at[pl.ds(s * DPT, DPT)])
    plsc.subcore_barrier()

    sems = [s0, s1, s2, s3]
    for p in range(4):
        pltpu.async_copy(ones_v, dacc.at[didx.at[p]], sems[p], add=True)

    def step(t, carry):
        for p in range(4):
            jj = 4 * t + p
            pltpu.make_async_copy(ones_v, dacc.at[didx.at[jj - 4]],
                                  sems[p]).wait()
            pltpu.async_copy(ones_v, dacc.at[didx.at[jj]], sems[p], add=True)
        return carry

    lax.fori_loop(1, NCHUNK // 4, step, 0)
    for p in range(4):
        jj = NCHUNK - 4 + p
        pltpu.make_async_copy(ones_v, dacc.at[didx.at[jj]], sems[p]).wait()

    plsc.subcore_barrier()
    pltpu.sync_copy(dacc.at[pl.ds(s * DPT, DPT)],
                    out_hbm.at[c, pl.ds(s * DPT, DPT)])


def _sc_deg(dst4, ones_k, z_deg):
    return pl.kernel(
        _sc_deg_body,
        out_type=jax.ShapeDtypeStruct((NC, N_PAD), jnp.float32),
        mesh=plsc.VectorSubcoreMesh(**_MESH),
        scratch_types=[
            pltpu.VMEM((NCHUNK, K), jnp.int32),
            pltpu.VMEM((K,), jnp.float32),
            pltpu.VMEM_SHARED((N_PAD,), jnp.float32),
            pltpu.SemaphoreType.DMA,
            pltpu.SemaphoreType.DMA,
            pltpu.SemaphoreType.DMA,
            pltpu.SemaphoreType.DMA,
        ],
        name="sc_gcn_deg",
    )(dst4, ones_k, z_deg)


def _sc_scatter_body(y_hbm, src_hbm, dst_hbm, zer_hbm, out_hbm,
                     sidx, didx,
                     b0, b1, b2, b3, b4, b5, b6, b7, acc,
                     g0, g1, g2, g3, g4, g5, g6, g7,
                     s0, s1, s2, s3, s4, s5, s6, s7):
    c = lax.axis_index("c")
    s = lax.axis_index("s")
    pltpu.sync_copy(src_hbm.at[c, s], sidx)

    bufs = [b0, b1, b2, b3, b4, b5, b6, b7]
    gs = [g0, g1, g2, g3, g4, g5, g6, g7]
    ss = [s0, s1, s2, s3, s4, s5, s6, s7]

    def g_start(jj, u):
        pltpu.async_copy(y_hbm.at[sidx.at[pl.ds(jj * KS, KS)]], bufs[u], gs[u])

    def g_wait(jj, u):
        pltpu.make_async_copy(y_hbm.at[sidx.at[pl.ds(jj * KS, KS)]], bufs[u], gs[u]).wait()

    def s_start(jj, u):
        pltpu.async_copy(bufs[u], acc.at[didx.at[pl.ds(jj * KS, KS)]], ss[u], add=True)

    def s_wait(jj, u):
        pltpu.make_async_copy(bufs[u], acc.at[didx.at[pl.ds(jj * KS, KS)]], ss[u]).wait()

    pltpu.sync_copy(dst_hbm.at[c, s], didx)
    pltpu.sync_copy(zer_hbm, acc.at[pl.ds(s * HPT, HPT), :])
    plsc.subcore_barrier()

    for u in range(4):
        g_start(u, u)
    for u in range(8):
        g_wait(u, u)
        s_start(u, u)
        if u >= 4:
            s_wait(u - 4, u - 4)
        g_start(u + 4, (u + 4) % NB)

    def step(t, carry):
        for u in range(8):
            jj = NB * t + u
            g_wait(jj, u)
            s_start(jj, u)
            s_wait(jj - 4, (u + 4) % NB)

            @pl.when(jj + 4 < NCHS)
            def _():
                g_start(jj + 4, (u + 4) % NB)
        return carry

    lax.fori_loop(1, NCHS // NB, step, 0)
    for u in range(4, 8):
        s_wait(NCHS - NB + u, u)
    plsc.subcore_barrier()
    pltpu.sync_copy(acc.at[pl.ds(s * OPT, OPT), :],
                    out_hbm.at[c, pl.ds(s * OPT, OPT), :])


def _sc_scatter(y, src4, dst4h, zrows):
    return pl.kernel(
        _sc_scatter_body,
        out_type=jax.ShapeDtypeStruct((NC, L, D), jnp.float32),
        mesh=plsc.VectorSubcoreMesh(**_MESH),
        scratch_types=(
            [pltpu.VMEM((EPT_PAD,), jnp.int32),
             pltpu.VMEM((EPT_PAD,), jnp.int32)]
            + [pltpu.VMEM((KS, D), jnp.float32) for _ in range(NB)]
            + [pltpu.VMEM_SHARED((H_PAD, D), jnp.float32)]
            + [pltpu.SemaphoreType.DMA for _ in range(2 * NB)]
        ),
        name="sc_gcn_scatter",
    )(y, src4, dst4h, zrows)


R = 1000


def _tc_prep_body(x_ref, d0_ref, d1_ref, g_ref, dinv_ref):
    dinv = lax.rsqrt(d0_ref[...] + d1_ref[...] + 1.0)
    g_ref[...] = x_ref[...] * dinv
    dinv_ref[...] = dinv


def _tc_prep(x, d0, d1):
    return pl.pallas_call(
        _tc_prep_body,
        grid=(N // R,),
        in_specs=[
            pl.BlockSpec((R, D_IN), lambda i: (i, 0)),
            pl.BlockSpec((R, 1), lambda i: (i, 0)),
            pl.BlockSpec((R, 1), lambda i: (i, 0)),
        ],
        out_specs=[
            pl.BlockSpec((R, D_IN), lambda i: (i, 0)),
            pl.BlockSpec((R, 1), lambda i: (i, 0)),
        ],
        out_shape=[
            jax.ShapeDtypeStruct((N, D_IN), jnp.float32),
            jax.ShapeDtypeStruct((N, 1), jnp.float32),
        ],
    )(x, d0, d1)


def _tc_mid_body(acc_ref, g_ref, dinv_ref, b0_ref, w0_ref, w1_ref, y1_ref):
    dinv = dinv_ref[...]
    agg = acc_ref[0] + acc_ref[1] + g_ref[...]
    pre = jnp.dot(agg, w0_ref[...],
                  preferred_element_type=jnp.float32) * dinv + b0_ref[...]
    h = jnp.maximum(pre, 0.0)
    y1_ref[...] = jnp.dot(h, w1_ref[...],
                          preferred_element_type=jnp.float32) * dinv


def _tc_mid(accp, g, dinv, b0, w0, w1):
    return pl.pallas_call(
        _tc_mid_body,
        grid=(N // R,),
        in_specs=[
            pl.BlockSpec((NC, R, D_IN), lambda i: (0, i, 0)),
            pl.BlockSpec((R, D_IN), lambda i: (i, 0)),
            pl.BlockSpec((R, 1), lambda i: (i, 0)),
            pl.BlockSpec((1, D_H), lambda i: (0, 0)),
            pl.BlockSpec((D_IN, D_H), lambda i: (0, 0)),
            pl.BlockSpec((D_H, D_OUT), lambda i: (0, 0)),
        ],
        out_specs=pl.BlockSpec((R, D_OUT), lambda i: (i, 0)),
        out_shape=jax.ShapeDtypeStruct((N, D_OUT), jnp.float32),
    )(accp, g, dinv, b0, w0, w1)


def _tc_final_body(acc_ref, y1_ref, dinv_ref, b1_ref, out_ref):
    pre = (acc_ref[0] + acc_ref[1] + y1_ref[...]) * dinv_ref[...] + b1_ref[...]
    out_ref[...] = jnp.maximum(pre, 0.0)


def _tc_final(accp, y1, dinv, b1):
    return pl.pallas_call(
        _tc_final_body,
        grid=(N // R,),
        in_specs=[
            pl.BlockSpec((NC, R, D_OUT), lambda i: (0, i, 0)),
            pl.BlockSpec((R, D_OUT), lambda i: (i, 0)),
            pl.BlockSpec((R, 1), lambda i: (i, 0)),
            pl.BlockSpec((1, D_OUT), lambda i: (0, 0)),
        ],
        out_specs=pl.BlockSpec((R, D_OUT), lambda i: (i, 0)),
        out_shape=jax.ShapeDtypeStruct((N, D_OUT), jnp.float32),
    )(accp, y1, dinv, b1)


@jax.jit
def kernel(x, edge_index, W0, b0, W1, b1):
    src = edge_index[0].astype(jnp.int32)
    dst = edge_index[1].astype(jnp.int32)
    pad_width = ((0, 0), (0, 0), (0, EPT_PAD - EPT))
    src4 = jnp.pad(src.reshape(NC, NS, EPT), pad_width)
    dst4 = dst.reshape(NC, NS, NCHUNK, K)
    fake = N_PAD + (jnp.arange(EPT_PAD - EPT, dtype=jnp.int32) % 128)
    dst3 = jnp.pad(dst.reshape(NC, NS, EPT), pad_width)
    dst3 = dst3.at[:, :, EPT:].set(fake[None, None, :])
    dump = L + (dst3 & 127)
    dstP = [jnp.where((dst3 >= p * L) & (dst3 < (p + 1) * L), dst3 - p * L,
                      dump) for p in range(NP)]
    ones_k = jnp.ones((K,), jnp.float32)
    z_deg = jnp.zeros((DPT,), jnp.float32)
    zrows = jnp.zeros((HPT, D), jnp.float32)

    degp = _sc_deg(dst4, ones_k, z_deg)
    d0 = degp[0, :N].reshape(N, 1)
    d1 = degp[1, :N].reshape(N, 1)

    g, dinv = _tc_prep(x, d0, d1)
    accp0 = jnp.concatenate(
        [_sc_scatter(g, src4, dstP[p], zrows) for p in range(NP)], axis=1)
    y1 = _tc_mid(accp0, g, dinv, b0.reshape(1, D_H), W0, W1)
    accp1 = jnp.concatenate(
        [_sc_scatter(y1, src4, dstP[p], zrows) for p in range(NP)], axis=1)
    return _tc_final(accp1, y1, dinv, b1.reshape(1, D_OUT))
